# Initial kernel scaffold; baseline (speedup 1.0000x reference)
import jax
import jax.numpy as jnp
from jax.experimental import pallas as pl


def kernel(xs_pad, conv_w, proj, codebook):
    raise NotImplementedError("write your pallas kernel here")


# speedup vs baseline: 15.8226x; 15.8226x over previous
import numpy as np
import jax
import jax.numpy as jnp
from jax.experimental import pallas as pl

SR = 16000
FRAME_LEN = 400
FRAME_SHIFT = 160
NFFT = 512
NMEL = 128
FBANK_MEAN = 15.41663
FBANK_STD = 6.55582
EMBED_DIM = 512
PATCH = 16
QUANT_N = 1024
QUANT_DIM = 256

_BINS = NFFT // 2 + 1
_KPAD = 384
_NFRM = 992
_NPATCH = 496
_ROWS = 1000
_EPS_LOG = 1.1920928955078125e-07

_C128 = np.float32(-0.0490873866)
_C512 = np.float32(-0.0122718466)
_C4 = np.float32(-1.57079637)

F32 = jnp.float32
U32 = jnp.uint32
BF = jnp.bfloat16
_HI = jax.lax.Precision.HIGHEST


def _build_np_consts():
    def hz2mel(h):
        return 1127.0 * np.log(1.0 + h / 700.0)

    low, high = 20.0, SR / 2.0
    mel_pts = np.linspace(hz2mel(low), hz2mel(high), NMEL + 2)
    fft_mel = hz2mel(np.arange(_BINS) * SR / NFFT)
    fb = np.zeros((NMEL, _BINS), dtype=np.float32)
    for m in range(NMEL):
        l, c, r = mel_pts[m], mel_pts[m + 1], mel_pts[m + 2]
        up = (fft_mel - l) / (c - l)
        down = (r - fft_mel) / (r - c)
        fb[m] = np.maximum(0.0, np.minimum(up, down)).astype(np.float32)
    melt = np.zeros((_KPAD, NMEL), dtype=np.float32)
    melt[:_BINS, :] = fb.T

    n = FRAME_LEN
    win = (0.5 - 0.5 * np.cos(2 * np.pi * np.arange(n) / (n - 1))) ** 0.85
    D = np.eye(n) - np.ones((n, n)) / n
    P = np.eye(n)
    P[0, 0] = 1.0 - 0.97
    P[np.arange(1, n), np.arange(0, n - 1)] = -0.97
    M = (2.0 ** 15) * (D.T @ P.T @ np.diag(win))
    mpre = np.zeros((3 * FRAME_SHIFT, NFFT), dtype=np.float64)
    mpre[:n, :n] = M
    fold = np.zeros((3 * FRAME_SHIFT, 4 * 128), dtype=np.float32)
    for b in range(4):
        fold[:, 128 * b:128 * (b + 1)] = mpre[:, b::4].astype(np.float32)

    mask = (np.arange(NMEL)[None, :] // PATCH == np.arange(8)[:, None])
    return fold, melt, mask.astype(np.float32)


_FOLD, _MELT, _MASK = _build_np_consts()


def _trig_tables():
    ii = jax.lax.broadcasted_iota(U32, (128, 128), 0)
    jj = jax.lax.broadcasted_iota(U32, (128, 128), 1)
    ang1 = ((ii * jj) & np.uint32(127)).astype(F32) * _C128
    c1 = jnp.cos(ang1)
    smc = jnp.sin(ang1) - c1

    ki = jax.lax.broadcasted_iota(U32, (128, 4), 0)
    bi = jax.lax.broadcasted_iota(U32, (128, 4), 1)
    angt = (ki * bi).astype(F32) * _C512
    tt = jnp.concatenate([jnp.cos(angt).T, jnp.sin(angt).T], axis=0)

    i4 = jax.lax.broadcasted_iota(U32, (4, 4), 0)
    j4 = jax.lax.broadcasted_iota(U32, (4, 4), 1)
    ang4 = ((i4 * j4) & np.uint32(3)).astype(F32) * _C4
    d4c = jnp.cos(ang4)
    d4s = jnp.sin(ang4)
    pad = jnp.zeros((8, 128), F32)
    d4c_p = pad.at[:4, :4].set(d4c)
    d4cs_p = pad.at[:4, :4].set(d4c + d4s)
    d4sc_p = pad.at[:4, :4].set(d4s - d4c)
    return c1, smc, tt, d4c_p, d4cs_p, d4sc_p


def _dot(a, b):
    return jax.lax.dot_general(a, b, (((1,), (0,)), ((), ())),
                               precision=_HI, preferred_element_type=F32)


def _dot_bf(a, b):
    return jax.lax.dot_general(a.astype(BF), b.astype(BF),
                               (((1,), (0,)), ((), ())),
                               preferred_element_type=F32)


def _tree4(parts):
    return (parts[0] + parts[1]) + (parts[2] + parts[3])


def _tok_kernel(x_ref, fold_ref, c1_ref, smc_ref, tt_ref, d4c_ref, d4cs_ref,
                d4sc_ref, melt_ref, mask_ref, wstack_ref, proj_ref, cbt_ref,
                out_ref):
    w2 = x_ref[0]
    c1 = c1_ref[...]
    smc = smc_ref[...]

    zre, zim, zsum = [], [], []
    for b in range(4):
        fb_cols = fold_ref[:, 128 * b:128 * (b + 1)]
        f_b = (_dot(w2[0:_NFRM], fb_cols[0:160])
               + _dot(w2[1:_NFRM + 1], fb_cols[160:320])
               + _dot(w2[2:_NFRM + 2], fb_cols[320:480]))
        a_b = _dot(f_b, c1)
        aim_b = a_b + _dot(f_b, smc)
        tc_b = tt_ref[b:b + 1, :]
        ts_b = tt_ref[4 + b:5 + b, :]
        zre_b = a_b * tc_b - aim_b * ts_b
        zim_b = a_b * ts_b + aim_b * tc_b
        zre.append(zre_b)
        zim.append(zim_b)
        zsum.append(zre_b + zim_b)

    powers = []
    for k2 in range(3):
        b1 = _tree4([zsum[b] * d4c_ref[b, k2] for b in range(4)])
        xr = b1 - _tree4([zim[b] * d4cs_ref[b, k2] for b in range(4)])
        xi = b1 + _tree4([zre[b] * d4sc_ref[b, k2] for b in range(4)])
        absre = jnp.abs(xr)
        absim = jnp.abs(xi)
        mx = jnp.maximum(absre, absim)
        mn = jnp.minimum(absre, absim)
        q = mn / mx
        h = mx * jnp.sqrt(1.0 + q * q)
        sel = jnp.where(h == h, h, mn)
        powers.append(sel * sel)
    lane = jax.lax.broadcasted_iota(jnp.int32, (1, 128), 1)
    bin256 = jnp.where(lane == 0, powers[2], 0.0)
    power = jnp.concatenate([powers[0], powers[1], bin256], axis=1)

    mel = _dot_bf(power, melt_ref[...])
    fb = (jnp.log(jnp.maximum(mel, _EPS_LOG)) - FBANK_MEAN) / (2.0 * FBANK_STD)

    fb3 = fb.reshape(62, PATCH, NMEL)
    mask = mask_ref[...]
    parts = []
    for h in range(PATCH):
        g = fb3[:, h, :]
        gb = jnp.broadcast_to(g[:, None, :], (62, 8, NMEL)) * mask[None]
        parts.append(gb.reshape(_NPATCH, NMEL))
    lhs = jnp.concatenate(parts, axis=1)
    feats = _dot_bf(lhs, wstack_ref[...])

    mu = jnp.mean(feats, axis=-1, keepdims=True)
    var = jnp.mean((feats - mu) ** 2, axis=-1, keepdims=True)
    feats = (feats - mu) / jnp.sqrt(var + 1e-5)

    v = _dot_bf(feats, proj_ref[...])
    v = v / (jnp.sqrt(jnp.sum(v * v, axis=-1, keepdims=True)) + 1e-12)

    cbt = cbt_ref[...]
    cnorm = jnp.sqrt(jnp.sum(cbt * cbt, axis=0, keepdims=True))
    cnt = cbt / (cnorm + 1e-12)
    csq = jnp.sum(cnt * cnt, axis=0, keepdims=True)
    s = _dot_bf(v, cnt)
    dist = csq - 2.0 * s
    idx = jnp.argmin(dist, axis=-1).astype(jnp.int32)
    out_ref[...] = idx.reshape(1, 1, _NPATCH)


def kernel(xs_pad, conv_w, proj, codebook):
    B = xs_pad.shape[0]
    x2 = xs_pad.reshape(B, _ROWS, FRAME_SHIFT)
    w = conv_w.reshape(EMBED_DIM, PATCH, PATCH)
    wstack = jnp.tile(w.transpose(1, 2, 0), (1, NMEL // PATCH, 1))
    wstack = wstack.reshape(PATCH * NMEL, EMBED_DIM)
    cbt = codebook.T
    c1, smc, tt, d4c, d4cs, d4sc = _trig_tables()

    full = lambda shape: pl.BlockSpec(shape, lambda i: tuple(0 for _ in shape))
    out = pl.pallas_call(
        _tok_kernel,
        grid=(B,),
        in_specs=[
            pl.BlockSpec((1, _ROWS, FRAME_SHIFT), lambda i: (i, 0, 0)),
            full((3 * FRAME_SHIFT, 512)),
            full((128, 128)),
            full((128, 128)),
            full((8, 128)),
            full((8, 128)),
            full((8, 128)),
            full((8, 128)),
            full((_KPAD, NMEL)),
            full((8, NMEL)),
            full((PATCH * NMEL, EMBED_DIM)),
            full((EMBED_DIM, QUANT_DIM)),
            full((QUANT_DIM, QUANT_N)),
        ],
        out_specs=pl.BlockSpec((1, 1, _NPATCH), lambda i: (i, 0, 0)),
        out_shape=jax.ShapeDtypeStruct((B, 1, _NPATCH), jnp.int32),
    )(x2, jnp.asarray(_FOLD), c1, smc, tt, d4c, d4cs, d4sc,
      jnp.asarray(_MELT), jnp.asarray(_MASK), wstack, proj, cbt)
    return out.reshape(B, _NPATCH)
